# baseline (speedup 1.0000x reference)
import functools

import jax
import jax.numpy as jnp
from jax import lax
from jax.experimental import pallas as pl
from jax.experimental.pallas import tpu as pltpu
from jax.experimental.pallas import tpu_sc as plsc

_NUM_CORES = 2
_NUM_SUBCORES = 16
_NW = _NUM_CORES * _NUM_SUBCORES
_CHUNK = 128


def _make_sc_kernel(n_chunks, chunk, seq, hidden, total):
  per_w = n_chunks * chunk
  mesh = plsc.VectorSubcoreMesh(core_axis_name="c", subcore_axis_name="s")

  @functools.partial(
      pl.kernel,
      out_type=jax.ShapeDtypeStruct((total, hidden), jnp.float32),
      mesh=mesh,
      scratch_types=[
          pltpu.VMEM((n_chunks, chunk), jnp.int32),
          pltpu.VMEM((2 * seq, hidden), jnp.float32),
          pltpu.VMEM((chunk, hidden), jnp.float32),
          pltpu.SemaphoreType.DMA,
      ],
      compiler_params=pltpu.CompilerParams(use_tc_tiling_on_sc=False),
  )
  def k(idx_hbm, table_hbm, pos2_hbm, out_hbm, idx_v, pos_v, rows_v, sem):
    wid = lax.axis_index("s") * _NUM_CORES + lax.axis_index("c")
    pltpu.sync_copy(idx_hbm.at[wid], idx_v)
    pltpu.sync_copy(pos2_hbm, pos_v)

    def body(j, carry):
      pltpu.async_copy(table_hbm.at[idx_v.at[j]], rows_v, sem).wait()
      p = lax.rem(j * chunk, seq)

      def add_row(i, c2):
        for c in range(hidden // 16):
          sl = pl.ds(c * 16, 16)
          plsc.addupdate(rows_v.at[i, sl], pos_v[p + i, sl])
        return c2

      lax.fori_loop(0, chunk, add_row, 0)
      pltpu.sync_copy(
          rows_v, out_hbm.at[pl.ds(wid * per_w + j * chunk, chunk)])
      return carry

    lax.fori_loop(0, n_chunks, body, 0)

  return k


def kernel(token_ids, token_embed, position_embed):
  batch, seq = token_ids.shape
  _, hidden = token_embed.shape
  total = batch * seq
  n_chunks = total // (_NW * _CHUNK)

  idx3 = token_ids.reshape(_NW, n_chunks, _CHUNK).astype(jnp.int32)
  pos = position_embed[0, :seq].astype(jnp.float32)
  pos2 = jnp.concatenate([pos, pos], axis=0)

  k = _make_sc_kernel(n_chunks, _CHUNK, seq, hidden, total)
  out = k(idx3, token_embed, pos2)
  return out.reshape(batch, seq, hidden)


# speedup vs baseline: 1.4057x; 1.4057x over previous
import functools

import jax
import jax.numpy as jnp
from jax import lax
from jax.experimental import pallas as pl
from jax.experimental.pallas import tpu as pltpu
from jax.experimental.pallas import tpu_sc as plsc

_NUM_CORES = 2
_NUM_SUBCORES = 16
_NW = _NUM_CORES * _NUM_SUBCORES
_CHUNK = 128
_NBUF = 4


def _make_sc_kernel(n_chunks, chunk, seq, hidden, total):
  per_w = n_chunks * chunk
  mesh = plsc.VectorSubcoreMesh(core_axis_name="c", subcore_axis_name="s")

  @functools.partial(
      pl.kernel,
      out_type=jax.ShapeDtypeStruct((total, hidden), jnp.float32),
      mesh=mesh,
      scratch_types=[
          pltpu.VMEM((n_chunks, chunk), jnp.int32),
          pltpu.VMEM((2 * seq, hidden), jnp.float32),
          pltpu.VMEM((_NBUF, chunk, hidden), jnp.float32),
          pltpu.SemaphoreType.DMA((_NBUF,)),
      ],
      compiler_params=pltpu.CompilerParams(use_tc_tiling_on_sc=False),
  )
  def k(idx_hbm, table_hbm, pos2_hbm, out_hbm, idx_v, pos_v, rows_v, sems):
    wid = lax.axis_index("s") * _NUM_CORES + lax.axis_index("c")
    pltpu.sync_copy(idx_hbm.at[wid], idx_v)
    pltpu.sync_copy(pos2_hbm, pos_v)

    for b in range(_NBUF):
      pltpu.async_copy(table_hbm.at[idx_v.at[b]], rows_v.at[b], sems.at[b])

    def outer(t, carry):
      for b in range(_NBUF):
        jj = t * _NBUF + b
        pltpu.make_async_copy(
            table_hbm.at[idx_v.at[jj]], rows_v.at[b], sems.at[b]).wait()
        p = lax.rem(jj * chunk, seq)

        @plsc.parallel_loop(0, chunk, unroll=8)
        def add_row(i):
          for c in range(hidden // 16):
            sl = pl.ds(c * 16, 16)
            plsc.addupdate(rows_v.at[b, i, sl], pos_v[p + i, sl])

        pltpu.sync_copy(
            rows_v.at[b], out_hbm.at[pl.ds(wid * per_w + jj * chunk, chunk)])

        nj = jj + _NBUF

        @pl.when(nj < n_chunks)
        def _():
          pltpu.async_copy(
              table_hbm.at[idx_v.at[nj]], rows_v.at[b], sems.at[b])

      return carry

    lax.fori_loop(0, n_chunks // _NBUF, outer, 0)

  return k


def kernel(token_ids, token_embed, position_embed):
  batch, seq = token_ids.shape
  _, hidden = token_embed.shape
  total = batch * seq
  n_chunks = total // (_NW * _CHUNK)

  idx3 = token_ids.reshape(_NW, n_chunks, _CHUNK).astype(jnp.int32)
  pos = position_embed[0, :seq].astype(jnp.float32)
  pos2 = jnp.concatenate([pos, pos], axis=0)

  k = _make_sc_kernel(n_chunks, _CHUNK, seq, hidden, total)
  out = k(idx3, token_embed, pos2)
  return out.reshape(batch, seq, hidden)


# speedup vs baseline: 1.8541x; 1.3190x over previous
import functools

import jax
import jax.numpy as jnp
from jax import lax
from jax.experimental import pallas as pl
from jax.experimental.pallas import tpu as pltpu
from jax.experimental.pallas import tpu_sc as plsc

_NUM_CORES = 2
_NUM_SUBCORES = 16
_NW = _NUM_CORES * _NUM_SUBCORES
_CHUNK = 128
_NBUF = 4


def _make_sc_kernel(n_chunks, chunk, seq, hidden, total):
  per_w = n_chunks * chunk
  mesh = plsc.VectorSubcoreMesh(core_axis_name="c", subcore_axis_name="s")

  @functools.partial(
      pl.kernel,
      out_type=jax.ShapeDtypeStruct((total, 128), jnp.float32),
      mesh=mesh,
      scratch_types=[
          pltpu.VMEM((n_chunks, chunk), jnp.int32),
          pltpu.VMEM((2 * seq, hidden), jnp.float32),
          pltpu.VMEM((_NBUF, chunk, hidden), jnp.float32),
          pltpu.SemaphoreType.DMA((_NBUF,)),
      ],
      compiler_params=pltpu.CompilerParams(use_tc_tiling_on_sc=False),
  )
  def k(idx_hbm, table_hbm, pos2_hbm, out_hbm, idx_v, pos_v, rows_v, sems):
    wid = lax.axis_index("s") * _NUM_CORES + lax.axis_index("c")
    pltpu.sync_copy(idx_hbm.at[wid], idx_v)
    pltpu.sync_copy(pos2_hbm, pos_v)

    for b in range(_NBUF):
      pltpu.async_copy(table_hbm.at[idx_v.at[b]], rows_v.at[b], sems.at[b])

    def outer(t, carry):
      for b in range(_NBUF):
        jj = t * _NBUF + b
        pltpu.make_async_copy(
            table_hbm.at[idx_v.at[jj]], rows_v.at[b], sems.at[b]).wait()
        p = lax.rem(jj * chunk, seq)

        @plsc.parallel_loop(0, chunk, unroll=8)
        def add_row(i):
          for c in range(hidden // 16):
            sl = pl.ds(c * 16, 16)
            plsc.addupdate(rows_v.at[b, i, sl], pos_v[p + i, sl])

        pltpu.sync_copy(
            rows_v.at[b],
            out_hbm.at[pl.ds(wid * per_w + jj * chunk, chunk),
                       pl.ds(0, hidden)])

        nj = jj + _NBUF

        @pl.when(nj < n_chunks)
        def _():
          pltpu.async_copy(
              table_hbm.at[idx_v.at[nj]], rows_v.at[b], sems.at[b])

      return carry

    lax.fori_loop(0, n_chunks // _NBUF, outer, 0)

  return k


def kernel(token_ids, token_embed, position_embed):
  batch, seq = token_ids.shape
  _, hidden = token_embed.shape
  total = batch * seq
  n_chunks = total // (_NW * _CHUNK)

  idx3 = token_ids.reshape(_NW, n_chunks, _CHUNK).astype(jnp.int32)
  pos = position_embed[0, :seq].astype(jnp.float32)
  pos2 = jnp.concatenate([pos, pos], axis=0)

  k = _make_sc_kernel(n_chunks, _CHUNK, seq, hidden, total)
  out = k(idx3, token_embed, pos2)
  return out[:, :hidden].reshape(batch, seq, hidden)
